# Initial kernel scaffold; baseline (speedup 1.0000x reference)
import jax
import jax.numpy as jnp
from jax.experimental import pallas as pl


def kernel(h, x_s, edge_index, edge_features, W1, b1, W2, b2):
    raise NotImplementedError("write your pallas kernel here")


# speedup vs baseline: 1.5185x; 1.5185x over previous
import functools
import math

import jax
import jax.numpy as jnp
from jax import lax
from jax.experimental import pallas as pl
from jax.experimental.pallas import tpu as pltpu

N = 10000
E = 320000
H = 128
S = 128
EF = 16
WID = 2 * H
IN_SIZE = 2 * H + 2 * S + EF

SCORE_BLK = 8000


def _pq_body(h_ref, xs_ref, w1a_ref, w1b_ref, w1c_ref, w1d_ref, b1_ref,
             p_ref, q_ref):
    h = h_ref[...]
    xs = xs_ref[...]
    p_ref[...] = (
        jnp.dot(h, w1a_ref[...], preferred_element_type=jnp.float32)
        + jnp.dot(xs, w1c_ref[...], preferred_element_type=jnp.float32)
    )
    q_ref[...] = (
        jnp.dot(h, w1b_ref[...], preferred_element_type=jnp.float32)
        + jnp.dot(xs, w1d_ref[...], preferred_element_type=jnp.float32)
        + b1_ref[...]
    )


def _score_body(ps_ref, qd_ref, ef_ref, w1e_ref, w2_ref, b2_ref, exp_ref):
    u = ps_ref[...] + qd_ref[...] + jnp.dot(
        ef_ref[...], w1e_ref[...], preferred_element_type=jnp.float32)
    hidden = jnp.maximum(u, 0.0)
    raw = jnp.dot(hidden, w2_ref[...], preferred_element_type=jnp.float32)
    raw = raw + b2_ref[0, 0]
    raw = jnp.where(raw >= 0.0, raw, 0.01 * raw)
    s = raw * (1.0 / math.sqrt(float(H)))
    exp_ref[...] = jnp.exp(s)


def kernel(h, x_s, edge_index, edge_features, W1, b1, W2, b2):
    src = edge_index[0]
    dst = edge_index[1]

    w1a = W1[:, 0:H].T
    w1b = W1[:, H:2 * H].T
    w1c = W1[:, 2 * H:2 * H + S].T
    w1d = W1[:, 2 * H + S:2 * H + 2 * S].T
    w1e = W1[:, 2 * H + 2 * S:].T

    p, q = pl.pallas_call(
        _pq_body,
        out_shape=(
            jax.ShapeDtypeStruct((N, WID), jnp.float32),
            jax.ShapeDtypeStruct((N, WID), jnp.float32),
        ),
    )(h, x_s, w1a, w1b, w1c, w1d, b1)

    ps = jnp.take(p, src, axis=0)
    qd = jnp.take(q, dst, axis=0)

    nblk = E // SCORE_BLK
    exp_s = pl.pallas_call(
        _score_body,
        grid=(nblk,),
        in_specs=[
            pl.BlockSpec((SCORE_BLK, WID), lambda i: (i, 0)),
            pl.BlockSpec((SCORE_BLK, WID), lambda i: (i, 0)),
            pl.BlockSpec((SCORE_BLK, EF), lambda i: (i, 0)),
            pl.BlockSpec((EF, WID), lambda i: (0, 0)),
            pl.BlockSpec((WID, 1), lambda i: (0, 0)),
            pl.BlockSpec((1, 1), lambda i: (0, 0)),
        ],
        out_specs=pl.BlockSpec((SCORE_BLK, 1), lambda i: (i, 0)),
        out_shape=jax.ShapeDtypeStruct((E, 1), jnp.float32),
    )(ps, qd, edge_features, w1e, W2.T, b2.reshape(1, 1))
    exp_s = exp_s[:, 0]

    z = jax.ops.segment_sum(exp_s, dst, num_segments=N)
    weights = exp_s / (z[dst] + 1e-9)
    agg = jax.ops.segment_sum(h[src] * weights[:, None], dst, num_segments=N)
    return (agg, weights)


# speedup vs baseline: 1.7068x; 1.1240x over previous
import functools
import math

import jax
import jax.numpy as jnp
from jax import lax
from jax.experimental import pallas as pl
from jax.experimental.pallas import tpu as pltpu
from jax.experimental.pallas import tpu_sc as plsc

N = 10000
E = 320000
H = 128
S = 128
EF = 16
WID = 2 * H
IN_SIZE = 2 * H + 2 * S + EF

NC = 2
NS = 16
NW = NC * NS
EW = E // NW
CA = 80
NCHUNK = EW // CA
NGRP = CA // 16
NZ = 10240
ZW = NZ // NS
W2P = WID + 16
RBLK = 8000
INV_TEMP = 1.0 / math.sqrt(float(H))

_mesh = plsc.VectorSubcoreMesh(core_axis_name="c", subcore_axis_name="s")


def _pq_body(h_ref, xs_ref, w1a_ref, w1b_ref, w1c_ref, w1d_ref, b1_ref,
             p_ref, q_ref):
    h = h_ref[...]
    xs = xs_ref[...]
    p_ref[...] = (
        jnp.dot(h, w1a_ref[...], preferred_element_type=jnp.float32)
        + jnp.dot(xs, w1c_ref[...], preferred_element_type=jnp.float32)
    )
    q_ref[...] = (
        jnp.dot(h, w1b_ref[...], preferred_element_type=jnp.float32)
        + jnp.dot(xs, w1d_ref[...], preferred_element_type=jnp.float32)
        + b1_ref[...]
    )


def _r_body(ef_ref, w1e_ref, r_ref):
    r_ref[...] = jnp.dot(ef_ref[...], w1e_ref[...],
                         preferred_element_type=jnp.float32)


def _final_body(o2_ref, out_ref):
    out_ref[...] = o2_ref[0, :N, :] + o2_ref[1, :N, :]


@functools.partial(
    pl.kernel,
    out_type=(
        jax.ShapeDtypeStruct((E,), jnp.float32),
        jax.ShapeDtypeStruct((NC, NZ), jnp.float32),
    ),
    mesh=_mesh,
    compiler_params=pltpu.CompilerParams(use_tc_tiling_on_sc=False, needs_layout_passes=False),
    scratch_types=[
        pltpu.VMEM((CA,), jnp.int32),
        pltpu.VMEM((CA,), jnp.int32),
        pltpu.VMEM((CA, WID), jnp.float32),
        pltpu.VMEM((CA, WID), jnp.float32),
        pltpu.VMEM((CA, WID), jnp.float32),
        pltpu.VMEM((W2P,), jnp.float32),
        pltpu.VMEM((CA,), jnp.float32),
        pltpu.VMEM((NZ,), jnp.float32),
        pltpu.VMEM((ZW,), jnp.float32),
        pltpu.VMEM((ZW,), jnp.float32),
        pltpu.VMEM_SHARED((NS, NZ), jnp.float32),
        pltpu.SemaphoreType.DMA,
        pltpu.SemaphoreType.DMA,
    ],
)
def _score_kernel(src_hbm, dst_hbm, p_hbm, q_hbm, r_hbm, w2_hbm,
                  exp_hbm, z2_hbm,
                  sidx, didx, pb, qb, rb, w2v, expb, zloc, zacc, ztmp,
                  zsh, semp, semq):
    cid = lax.axis_index("c")
    sid = lax.axis_index("s")
    wid = sid * NC + cid
    base0 = wid * EW

    pltpu.sync_copy(w2_hbm, w2v)
    b2s = w2v[pl.ds(WID, 16)]

    def _zero_zloc(i, carry):
        zloc[pl.ds(i * 16, 16)] = jnp.zeros((16,), jnp.float32)
        return carry
    lax.fori_loop(0, NZ // 16, _zero_zloc, 0)

    def _chunk(ci, carry):
        base = base0 + ci * CA
        pltpu.sync_copy(src_hbm.at[pl.ds(base, CA)], sidx)
        pltpu.sync_copy(dst_hbm.at[pl.ds(base, CA)], didx)
        cp = pltpu.async_copy(p_hbm.at[sidx], pb, semp)
        cq = pltpu.async_copy(q_hbm.at[didx], qb, semq)
        pltpu.sync_copy(r_hbm.at[pl.ds(base, CA)], rb)
        cp.wait()
        cq.wait()

        def _group(g, gcarry):
            eidx = g * 16 + lax.iota(jnp.int32, 16)
            zero = jnp.zeros((16,), jnp.float32)

            def _kstep(k4, accs):
                a0, a1, a2, a3 = accs
                outs = []
                for u, a in ((0, a0), (1, a1), (2, a2), (3, a3)):
                    k = k4 * 4 + u
                    kv = jnp.full((16,), k, jnp.int32)
                    pv = plsc.load_gather(pb, [eidx, kv])
                    qv = plsc.load_gather(qb, [eidx, kv])
                    rv = plsc.load_gather(rb, [eidx, kv])
                    hv = jnp.maximum(pv + qv + rv, 0.0)
                    w2k = plsc.load_gather(w2v, [kv])
                    outs.append(a + hv * w2k)
                return tuple(outs)

            a0, a1, a2, a3 = lax.fori_loop(
                0, WID // 4, _kstep, (zero, zero, zero, zero))
            raw = (a0 + a1) + (a2 + a3) + b2s
            raw = jnp.where(raw >= 0.0, raw, 0.01 * raw)
            es = jnp.exp(raw * INV_TEMP)
            expb[pl.ds(g * 16, 16)] = es
            didx_g = didx[pl.ds(g * 16, 16)]
            plsc.addupdate_scatter(zloc, [didx_g], es)
            return gcarry

        lax.fori_loop(0, NGRP, _group, 0)
        pltpu.sync_copy(expb, exp_hbm.at[pl.ds(base, CA)])
        return carry

    lax.fori_loop(0, NCHUNK, _chunk, 0)

    pltpu.sync_copy(zloc, zsh.at[sid])
    plsc.subcore_barrier()
    off = sid * ZW

    def _zero_zacc(i, carry):
        zacc[pl.ds(i * 16, 16)] = jnp.zeros((16,), jnp.float32)
        return carry
    lax.fori_loop(0, ZW // 16, _zero_zacc, 0)

    def _reduce(j, carry):
        pltpu.sync_copy(zsh.at[j, pl.ds(off, ZW)], ztmp)

        def _acc(i, c2):
            sl = pl.ds(i * 16, 16)
            zacc[sl] = zacc[sl] + ztmp[sl]
            return c2
        lax.fori_loop(0, ZW // 16, _acc, 0)
        return carry
    lax.fori_loop(0, NS, _reduce, 0)
    pltpu.sync_copy(zacc, z2_hbm.at[cid, pl.ds(off, ZW)])


@functools.partial(
    pl.kernel,
    out_type=(
        jax.ShapeDtypeStruct((E,), jnp.float32),
        jax.ShapeDtypeStruct((NC, NZ, H), jnp.float32),
    ),
    mesh=_mesh,
    compiler_params=pltpu.CompilerParams(use_tc_tiling_on_sc=False, needs_layout_passes=False),
    scratch_types=[
        pltpu.VMEM((CA,), jnp.int32),
        pltpu.VMEM((CA,), jnp.int32),
        pltpu.VMEM((CA, H), jnp.float32),
        pltpu.VMEM((CA,), jnp.float32),
        pltpu.VMEM((CA,), jnp.float32),
        pltpu.VMEM((NZ,), jnp.float32),
        pltpu.VMEM((NZ,), jnp.float32),
        pltpu.VMEM_SHARED((NZ, H), jnp.float32),
        pltpu.SemaphoreType.DMA,
    ],
)
def _agg_kernel(src_hbm, dst_hbm, exp_hbm, z2_hbm, h_hbm,
                w_hbm, o2_hbm,
                sidx, didx, hb, eb, wb, za, zb, osh, semh):
    cid = lax.axis_index("c")
    sid = lax.axis_index("s")
    wid = sid * NC + cid
    base0 = wid * EW

    pltpu.sync_copy(z2_hbm.at[0], za)
    pltpu.sync_copy(z2_hbm.at[1], zb)

    def _zsum(i, carry):
        sl = pl.ds(i * 16, 16)
        za[sl] = za[sl] + zb[sl]
        return carry
    lax.fori_loop(0, NZ // 16, _zsum, 0)

    def _zero_hb(e, carry):
        for j in range(H // 16):
            hb[e, pl.ds(j * 16, 16)] = jnp.zeros((16,), jnp.float32)
        return carry
    lax.fori_loop(0, CA, _zero_hb, 0)

    def _zero_osh(j, carry):
        pltpu.sync_copy(hb, osh.at[pl.ds(sid * ZW + j * CA, CA)])
        return carry
    lax.fori_loop(0, ZW // CA, _zero_osh, 0)
    plsc.subcore_barrier()

    def _chunk(ci, carry):
        base = base0 + ci * CA
        pltpu.sync_copy(src_hbm.at[pl.ds(base, CA)], sidx)
        pltpu.sync_copy(dst_hbm.at[pl.ds(base, CA)], didx)
        ch = pltpu.async_copy(h_hbm.at[sidx], hb, semh)
        pltpu.sync_copy(exp_hbm.at[pl.ds(base, CA)], eb)

        def _wgrp(g, gcarry):
            sl = pl.ds(g * 16, 16)
            didx_g = didx[sl]
            zv = plsc.load_gather(za, [didx_g])
            wb[sl] = eb[sl] / (zv + 1e-9)
            return gcarry
        lax.fori_loop(0, NGRP, _wgrp, 0)
        ch.wait()

        def _scale(e, scarry):
            we = plsc.load_gather(wb, [jnp.full((16,), e, jnp.int32)])
            for j in range(H // 16):
                sl = pl.ds(j * 16, 16)
                hb[e, sl] = hb[e, sl] * we
            return scarry
        lax.fori_loop(0, CA, _scale, 0)

        pltpu.sync_copy(wb, w_hbm.at[pl.ds(base, CA)])
        pltpu.sync_copy(hb, osh.at[didx], add=True)
        return carry

    lax.fori_loop(0, NCHUNK, _chunk, 0)
    plsc.subcore_barrier()
    pltpu.sync_copy(osh.at[pl.ds(sid * ZW, ZW)],
                    o2_hbm.at[cid, pl.ds(sid * ZW, ZW)])


def kernel(h, x_s, edge_index, edge_features, W1, b1, W2, b2):
    src = edge_index[0]
    dst = edge_index[1]

    w1a = W1[:, 0:H].T
    w1b = W1[:, H:2 * H].T
    w1c = W1[:, 2 * H:2 * H + S].T
    w1d = W1[:, 2 * H + S:2 * H + 2 * S].T
    w1e = W1[:, 2 * H + 2 * S:].T
    w2pad = jnp.concatenate(
        [W2.reshape(-1), jnp.broadcast_to(b2, (16,))]).astype(jnp.float32)

    p, q = pl.pallas_call(
        _pq_body,
        out_shape=(
            jax.ShapeDtypeStruct((N, WID), jnp.float32),
            jax.ShapeDtypeStruct((N, WID), jnp.float32),
        ),
    )(h, x_s, w1a, w1b, w1c, w1d, b1)

    r = pl.pallas_call(
        _r_body,
        grid=(E // RBLK,),
        in_specs=[
            pl.BlockSpec((RBLK, EF), lambda i: (i, 0)),
            pl.BlockSpec((EF, WID), lambda i: (0, 0)),
        ],
        out_specs=pl.BlockSpec((RBLK, WID), lambda i: (i, 0)),
        out_shape=jax.ShapeDtypeStruct((E, WID), jnp.float32),
    )(edge_features, w1e)

    exp_s, z2 = _score_kernel(src, dst, p, q, r, w2pad)
    weights, o2 = _agg_kernel(src, dst, exp_s, z2, h)

    agg = pl.pallas_call(
        _final_body,
        out_shape=jax.ShapeDtypeStruct((N, H), jnp.float32),
    )(o2)
    return (agg, weights)


# speedup vs baseline: 4.6243x; 2.7093x over previous
import functools
import math

import jax
import jax.numpy as jnp
from jax import lax
from jax.experimental import pallas as pl
from jax.experimental.pallas import tpu as pltpu
from jax.experimental.pallas import tpu_sc as plsc

N = 10000
E = 320000
H = 128
S = 128
EF = 16
WID = 2 * H
IN_SIZE = 2 * H + 2 * S + EF

NC = 2
NS = 16
NW = NC * NS
EW = E // NW
CA = 80
NCHUNK = EW // CA
NGRP = CA // 16
NZ = 10240
ZW = NZ // NS
W2P = WID + 16
RBLK = 8000
INV_TEMP = 1.0 / math.sqrt(float(H))

_mesh = plsc.VectorSubcoreMesh(core_axis_name="c", subcore_axis_name="s")


def _pq_body(h_ref, xs_ref, w1a_ref, w1b_ref, w1c_ref, w1d_ref, b1_ref,
             p_ref, q_ref):
    h = h_ref[...]
    xs = xs_ref[...]
    p_ref[...] = (
        jnp.dot(h, w1a_ref[...], preferred_element_type=jnp.float32)
        + jnp.dot(xs, w1c_ref[...], preferred_element_type=jnp.float32)
    )
    q_ref[...] = (
        jnp.dot(h, w1b_ref[...], preferred_element_type=jnp.float32)
        + jnp.dot(xs, w1d_ref[...], preferred_element_type=jnp.float32)
        + b1_ref[...]
    )


def _r_body(ef_ref, w1e_ref, r_ref):
    r_ref[...] = jnp.dot(ef_ref[...], w1e_ref[...],
                         preferred_element_type=jnp.float32)


def _final_body(o2_ref, out_ref):
    out_ref[...] = o2_ref[0, :N, :] + o2_ref[1, :N, :]


@functools.partial(
    pl.kernel,
    out_type=(
        jax.ShapeDtypeStruct((E,), jnp.float32),
        jax.ShapeDtypeStruct((NC, NZ), jnp.float32),
    ),
    mesh=_mesh,
    compiler_params=pltpu.CompilerParams(use_tc_tiling_on_sc=False, needs_layout_passes=False),
    scratch_types=[
        pltpu.VMEM((CA,), jnp.int32),
        pltpu.VMEM((CA,), jnp.int32),
        pltpu.VMEM((CA, WID), jnp.float32),
        pltpu.VMEM((CA, WID), jnp.float32),
        pltpu.VMEM((CA, WID), jnp.float32),
        pltpu.VMEM((W2P,), jnp.float32),
        pltpu.VMEM((CA,), jnp.float32),
        pltpu.VMEM((NZ,), jnp.float32),
        pltpu.VMEM((ZW,), jnp.float32),
        pltpu.VMEM((ZW,), jnp.float32),
        pltpu.VMEM_SHARED((NS, NZ), jnp.float32),
        pltpu.SemaphoreType.DMA,
        pltpu.SemaphoreType.DMA,
    ],
)
def _score_kernel(src_hbm, dst_hbm, p_hbm, q_hbm, r_hbm, w2_hbm,
                  exp_hbm, z2_hbm,
                  sidx, didx, pb, qb, rb, w2v, expb, zloc, zacc, ztmp,
                  zsh, semp, semq):
    cid = lax.axis_index("c")
    sid = lax.axis_index("s")
    wid = sid * NC + cid
    base0 = wid * EW

    pltpu.sync_copy(w2_hbm, w2v)
    b2s = w2v[pl.ds(WID, 16)]

    def _zero_zloc(i, carry):
        zloc[pl.ds(i * 16, 16)] = jnp.zeros((16,), jnp.float32)
        return carry
    lax.fori_loop(0, NZ // 16, _zero_zloc, 0)

    def _chunk(ci, carry):
        base = base0 + ci * CA
        pltpu.sync_copy(src_hbm.at[pl.ds(base, CA)], sidx)
        pltpu.sync_copy(dst_hbm.at[pl.ds(base, CA)], didx)
        cp = pltpu.async_copy(p_hbm.at[sidx], pb, semp)
        cq = pltpu.async_copy(q_hbm.at[didx], qb, semq)
        pltpu.sync_copy(r_hbm.at[pl.ds(base, CA)], rb)
        cp.wait()
        cq.wait()

        lane15 = lax.iota(jnp.int32, 16) == 15

        def _edge(e, ecarry):
            acc0 = jnp.zeros((16,), jnp.float32)
            acc1 = jnp.zeros((16,), jnp.float32)
            for j in range(WID // 16):
                sl = pl.ds(j * 16, 16)
                u = pb[e, sl] + qb[e, sl] + rb[e, sl]
                hv = jnp.maximum(u, 0.0) * w2v[sl]
                if j % 2 == 0:
                    acc0 = acc0 + hv
                else:
                    acc1 = acc1 + hv
            sv = plsc.cumsum(acc0 + acc1)
            plsc.store_scatter(expb, [jnp.full((16,), e, jnp.int32)], sv,
                               mask=lane15)
            return ecarry

        lax.fori_loop(0, CA, _edge, 0)

        def _group(g, gcarry):
            sl = pl.ds(g * 16, 16)
            raw = expb[sl] + b2s
            raw = jnp.where(raw >= 0.0, raw, 0.01 * raw)
            es = jnp.exp(raw * INV_TEMP)
            expb[sl] = es
            didx_g = didx[sl]
            plsc.addupdate_scatter(zloc, [didx_g], es)
            return gcarry

        lax.fori_loop(0, NGRP, _group, 0)
        pltpu.sync_copy(expb, exp_hbm.at[pl.ds(base, CA)])
        return carry

    lax.fori_loop(0, NCHUNK, _chunk, 0)

    pltpu.sync_copy(zloc, zsh.at[sid])
    plsc.subcore_barrier()
    off = sid * ZW

    def _zero_zacc(i, carry):
        zacc[pl.ds(i * 16, 16)] = jnp.zeros((16,), jnp.float32)
        return carry
    lax.fori_loop(0, ZW // 16, _zero_zacc, 0)

    def _reduce(j, carry):
        pltpu.sync_copy(zsh.at[j, pl.ds(off, ZW)], ztmp)

        def _acc(i, c2):
            sl = pl.ds(i * 16, 16)
            zacc[sl] = zacc[sl] + ztmp[sl]
            return c2
        lax.fori_loop(0, ZW // 16, _acc, 0)
        return carry
    lax.fori_loop(0, NS, _reduce, 0)
    pltpu.sync_copy(zacc, z2_hbm.at[cid, pl.ds(off, ZW)])


@functools.partial(
    pl.kernel,
    out_type=(
        jax.ShapeDtypeStruct((E,), jnp.float32),
        jax.ShapeDtypeStruct((NC, NZ, H), jnp.float32),
    ),
    mesh=_mesh,
    compiler_params=pltpu.CompilerParams(use_tc_tiling_on_sc=False, needs_layout_passes=False),
    scratch_types=[
        pltpu.VMEM((CA,), jnp.int32),
        pltpu.VMEM((CA,), jnp.int32),
        pltpu.VMEM((CA, H), jnp.float32),
        pltpu.VMEM((CA,), jnp.float32),
        pltpu.VMEM((CA,), jnp.float32),
        pltpu.VMEM((NZ,), jnp.float32),
        pltpu.VMEM((NZ,), jnp.float32),
        pltpu.VMEM_SHARED((NZ, H), jnp.float32),
        pltpu.SemaphoreType.DMA,
    ],
)
def _agg_kernel(src_hbm, dst_hbm, exp_hbm, z2_hbm, h_hbm,
                w_hbm, o2_hbm,
                sidx, didx, hb, eb, wb, za, zb, osh, semh):
    cid = lax.axis_index("c")
    sid = lax.axis_index("s")
    wid = sid * NC + cid
    base0 = wid * EW

    pltpu.sync_copy(z2_hbm.at[0], za)
    pltpu.sync_copy(z2_hbm.at[1], zb)

    def _zsum(i, carry):
        sl = pl.ds(i * 16, 16)
        za[sl] = za[sl] + zb[sl]
        return carry
    lax.fori_loop(0, NZ // 16, _zsum, 0)

    def _zero_hb(e, carry):
        for j in range(H // 16):
            hb[e, pl.ds(j * 16, 16)] = jnp.zeros((16,), jnp.float32)
        return carry
    lax.fori_loop(0, CA, _zero_hb, 0)

    def _zero_osh(j, carry):
        pltpu.sync_copy(hb, osh.at[pl.ds(sid * ZW + j * CA, CA)])
        return carry
    lax.fori_loop(0, ZW // CA, _zero_osh, 0)
    plsc.subcore_barrier()

    def _chunk(ci, carry):
        base = base0 + ci * CA
        pltpu.sync_copy(src_hbm.at[pl.ds(base, CA)], sidx)
        pltpu.sync_copy(dst_hbm.at[pl.ds(base, CA)], didx)
        ch = pltpu.async_copy(h_hbm.at[sidx], hb, semh)
        pltpu.sync_copy(exp_hbm.at[pl.ds(base, CA)], eb)

        def _wgrp(g, gcarry):
            sl = pl.ds(g * 16, 16)
            didx_g = didx[sl]
            zv = plsc.load_gather(za, [didx_g])
            wb[sl] = eb[sl] / (zv + 1e-9)
            return gcarry
        lax.fori_loop(0, NGRP, _wgrp, 0)
        ch.wait()

        def _scale(e, scarry):
            we = plsc.load_gather(wb, [jnp.full((16,), e, jnp.int32)])
            for j in range(H // 16):
                sl = pl.ds(j * 16, 16)
                hb[e, sl] = hb[e, sl] * we
            return scarry
        lax.fori_loop(0, CA, _scale, 0)

        pltpu.sync_copy(wb, w_hbm.at[pl.ds(base, CA)])
        pltpu.sync_copy(hb, osh.at[didx], add=True)
        return carry

    lax.fori_loop(0, NCHUNK, _chunk, 0)
    plsc.subcore_barrier()
    pltpu.sync_copy(osh.at[pl.ds(sid * ZW, ZW)],
                    o2_hbm.at[cid, pl.ds(sid * ZW, ZW)])


def kernel(h, x_s, edge_index, edge_features, W1, b1, W2, b2):
    src = edge_index[0]
    dst = edge_index[1]

    w1a = W1[:, 0:H].T
    w1b = W1[:, H:2 * H].T
    w1c = W1[:, 2 * H:2 * H + S].T
    w1d = W1[:, 2 * H + S:2 * H + 2 * S].T
    w1e = W1[:, 2 * H + 2 * S:].T
    w2pad = jnp.concatenate(
        [W2.reshape(-1), jnp.broadcast_to(b2, (16,))]).astype(jnp.float32)

    p, q = pl.pallas_call(
        _pq_body,
        out_shape=(
            jax.ShapeDtypeStruct((N, WID), jnp.float32),
            jax.ShapeDtypeStruct((N, WID), jnp.float32),
        ),
    )(h, x_s, w1a, w1b, w1c, w1d, b1)

    r = pl.pallas_call(
        _r_body,
        grid=(E // RBLK,),
        in_specs=[
            pl.BlockSpec((RBLK, EF), lambda i: (i, 0)),
            pl.BlockSpec((EF, WID), lambda i: (0, 0)),
        ],
        out_specs=pl.BlockSpec((RBLK, WID), lambda i: (i, 0)),
        out_shape=jax.ShapeDtypeStruct((E, WID), jnp.float32),
    )(edge_features, w1e)

    exp_s, z2 = _score_kernel(src, dst, p, q, r, w2pad)
    weights, o2 = _agg_kernel(src, dst, exp_s, z2, h)

    agg = pl.pallas_call(
        _final_body,
        out_shape=jax.ShapeDtypeStruct((N, H), jnp.float32),
    )(o2)
    return (agg, weights)


# speedup vs baseline: 5.6847x; 1.2293x over previous
import functools
import math

import jax
import jax.numpy as jnp
from jax import lax
from jax.experimental import pallas as pl
from jax.experimental.pallas import tpu as pltpu
from jax.experimental.pallas import tpu_sc as plsc

N = 10000
E = 320000
H = 128
S = 128
EF = 16
WID = 2 * H
IN_SIZE = 2 * H + 2 * S + EF

NC = 2
NS = 16
NW = NC * NS
EW = E // NW
CA = 80
NCHUNK = EW // CA
NGRP = CA // 16
NZ = 10240
ZW = NZ // NS
W2P = WID + 16
RBLK = 8000
INV_TEMP = 1.0 / math.sqrt(float(H))

_mesh = plsc.VectorSubcoreMesh(core_axis_name="c", subcore_axis_name="s")


def _pq_body(h_ref, xs_ref, w1a_ref, w1b_ref, w1c_ref, w1d_ref, b1_ref,
             p_ref, q_ref):
    h = h_ref[...]
    xs = xs_ref[...]
    p_ref[...] = (
        jnp.dot(h, w1a_ref[...], preferred_element_type=jnp.float32)
        + jnp.dot(xs, w1c_ref[...], preferred_element_type=jnp.float32)
    ).astype(jnp.bfloat16)
    q_ref[...] = (
        jnp.dot(h, w1b_ref[...], preferred_element_type=jnp.float32)
        + jnp.dot(xs, w1d_ref[...], preferred_element_type=jnp.float32)
        + b1_ref[...]
    ).astype(jnp.bfloat16)


def _r_body(ef_ref, w1e_ref, r_ref):
    r_ref[...] = jnp.dot(ef_ref[...], w1e_ref[...],
                         preferred_element_type=jnp.float32).astype(jnp.bfloat16)


def _final_body(o2_ref, out_ref):
    out_ref[...] = o2_ref[0, :N, :] + o2_ref[1, :N, :]


@functools.partial(
    pl.kernel,
    out_type=(
        jax.ShapeDtypeStruct((E,), jnp.float32),
        jax.ShapeDtypeStruct((NC, NZ), jnp.float32),
    ),
    mesh=_mesh,
    compiler_params=pltpu.CompilerParams(use_tc_tiling_on_sc=False, needs_layout_passes=False),
    scratch_types=[
        pltpu.VMEM((CA,), jnp.int32),
        pltpu.VMEM((CA,), jnp.int32),
        pltpu.VMEM((CA, WID), jnp.bfloat16),
        pltpu.VMEM((CA, WID), jnp.bfloat16),
        pltpu.VMEM((CA, WID), jnp.bfloat16),
        pltpu.VMEM((WID,), jnp.bfloat16),
        pltpu.VMEM((16,), jnp.float32),
        pltpu.VMEM((CA,), jnp.float32),
        pltpu.VMEM((NZ,), jnp.float32),
        pltpu.VMEM((ZW,), jnp.float32),
        pltpu.VMEM((ZW,), jnp.float32),
        pltpu.VMEM_SHARED((NS, NZ), jnp.float32),
        pltpu.SemaphoreType.DMA,
        pltpu.SemaphoreType.DMA,
    ],
)
def _score_kernel(src_hbm, dst_hbm, p_hbm, q_hbm, r_hbm, w2_hbm, b2_hbm,
                  exp_hbm, z2_hbm,
                  sidx, didx, pb, qb, rb, w2v, b2v, expb, zloc, zacc, ztmp,
                  zsh, semp, semq):
    cid = lax.axis_index("c")
    sid = lax.axis_index("s")
    wid = sid * NC + cid
    base0 = wid * EW

    pltpu.sync_copy(w2_hbm, w2v)
    pltpu.sync_copy(b2_hbm, b2v)
    b2s = b2v[...]

    def _zero_zloc(i, carry):
        zloc[pl.ds(i * 16, 16)] = jnp.zeros((16,), jnp.float32)
        return carry
    lax.fori_loop(0, NZ // 16, _zero_zloc, 0)

    def _chunk(ci, carry):
        base = base0 + ci * CA
        pltpu.sync_copy(src_hbm.at[pl.ds(base, CA)], sidx)
        pltpu.sync_copy(dst_hbm.at[pl.ds(base, CA)], didx)
        cp = pltpu.async_copy(p_hbm.at[sidx], pb, semp)
        cq = pltpu.async_copy(q_hbm.at[didx], qb, semq)
        pltpu.sync_copy(r_hbm.at[pl.ds(base, CA)], rb)
        cp.wait()
        cq.wait()

        lane15 = lax.iota(jnp.int32, 16) == 15

        def _edge(e, ecarry):
            acc0 = jnp.zeros((16,), jnp.float32)
            acc1 = jnp.zeros((16,), jnp.float32)
            bzero = jnp.zeros((32,), jnp.bfloat16)
            for j in range(WID // 32):
                sl = pl.ds(j * 32, 32)
                u = pb[e, sl] + qb[e, sl] + rb[e, sl]
                hv = jnp.maximum(u, bzero) * w2v[sl]
                t0, t1 = plsc.unpack(hv, format=plsc.PackFormat.INTERLEAVED)
                acc0 = acc0 + t0
                acc1 = acc1 + t1
            sv = plsc.cumsum(acc0 + acc1)
            plsc.store_scatter(expb, [jnp.full((16,), e, jnp.int32)], sv,
                               mask=lane15)
            return ecarry

        lax.fori_loop(0, CA, _edge, 0)

        def _group(g, gcarry):
            sl = pl.ds(g * 16, 16)
            raw = expb[sl] + b2s
            raw = jnp.where(raw >= 0.0, raw, 0.01 * raw)
            es = jnp.exp(raw * INV_TEMP)
            expb[sl] = es
            didx_g = didx[sl]
            plsc.addupdate_scatter(zloc, [didx_g], es)
            return gcarry

        lax.fori_loop(0, NGRP, _group, 0)
        pltpu.sync_copy(expb, exp_hbm.at[pl.ds(base, CA)])
        return carry

    lax.fori_loop(0, NCHUNK, _chunk, 0)

    pltpu.sync_copy(zloc, zsh.at[sid])
    plsc.subcore_barrier()
    off = sid * ZW

    def _zero_zacc(i, carry):
        zacc[pl.ds(i * 16, 16)] = jnp.zeros((16,), jnp.float32)
        return carry
    lax.fori_loop(0, ZW // 16, _zero_zacc, 0)

    def _reduce(j, carry):
        pltpu.sync_copy(zsh.at[j, pl.ds(off, ZW)], ztmp)

        def _acc(i, c2):
            sl = pl.ds(i * 16, 16)
            zacc[sl] = zacc[sl] + ztmp[sl]
            return c2
        lax.fori_loop(0, ZW // 16, _acc, 0)
        return carry
    lax.fori_loop(0, NS, _reduce, 0)
    pltpu.sync_copy(zacc, z2_hbm.at[cid, pl.ds(off, ZW)])


@functools.partial(
    pl.kernel,
    out_type=(
        jax.ShapeDtypeStruct((E,), jnp.float32),
        jax.ShapeDtypeStruct((NC, NZ, H), jnp.float32),
    ),
    mesh=_mesh,
    compiler_params=pltpu.CompilerParams(use_tc_tiling_on_sc=False, needs_layout_passes=False),
    scratch_types=[
        pltpu.VMEM((CA,), jnp.int32),
        pltpu.VMEM((CA,), jnp.int32),
        pltpu.VMEM((CA, H), jnp.float32),
        pltpu.VMEM((CA,), jnp.float32),
        pltpu.VMEM((CA,), jnp.float32),
        pltpu.VMEM((NZ,), jnp.float32),
        pltpu.VMEM((NZ,), jnp.float32),
        pltpu.VMEM_SHARED((NZ, H), jnp.float32),
        pltpu.SemaphoreType.DMA,
    ],
)
def _agg_kernel(src_hbm, dst_hbm, exp_hbm, z2_hbm, h_hbm,
                w_hbm, o2_hbm,
                sidx, didx, hb, eb, wb, za, zb, osh, semh):
    cid = lax.axis_index("c")
    sid = lax.axis_index("s")
    wid = sid * NC + cid
    base0 = wid * EW

    pltpu.sync_copy(z2_hbm.at[0], za)
    pltpu.sync_copy(z2_hbm.at[1], zb)

    def _zsum(i, carry):
        sl = pl.ds(i * 16, 16)
        za[sl] = za[sl] + zb[sl]
        return carry
    lax.fori_loop(0, NZ // 16, _zsum, 0)

    def _zero_hb(e, carry):
        for j in range(H // 16):
            hb[e, pl.ds(j * 16, 16)] = jnp.zeros((16,), jnp.float32)
        return carry
    lax.fori_loop(0, CA, _zero_hb, 0)

    def _zero_osh(j, carry):
        pltpu.sync_copy(hb, osh.at[pl.ds(sid * ZW + j * CA, CA)])
        return carry
    lax.fori_loop(0, ZW // CA, _zero_osh, 0)
    plsc.subcore_barrier()

    def _chunk(ci, carry):
        base = base0 + ci * CA
        pltpu.sync_copy(src_hbm.at[pl.ds(base, CA)], sidx)
        pltpu.sync_copy(dst_hbm.at[pl.ds(base, CA)], didx)
        ch = pltpu.async_copy(h_hbm.at[sidx], hb, semh)
        pltpu.sync_copy(exp_hbm.at[pl.ds(base, CA)], eb)

        def _wgrp(g, gcarry):
            sl = pl.ds(g * 16, 16)
            didx_g = didx[sl]
            zv = plsc.load_gather(za, [didx_g])
            wb[sl] = eb[sl] / (zv + 1e-9)
            return gcarry
        lax.fori_loop(0, NGRP, _wgrp, 0)
        ch.wait()

        def _scale(e, scarry):
            we = plsc.load_gather(wb, [jnp.full((16,), e, jnp.int32)])
            for j in range(H // 16):
                sl = pl.ds(j * 16, 16)
                hb[e, sl] = hb[e, sl] * we
            return scarry
        lax.fori_loop(0, CA, _scale, 0)

        pltpu.sync_copy(wb, w_hbm.at[pl.ds(base, CA)])
        pltpu.sync_copy(hb, osh.at[didx], add=True)
        return carry

    lax.fori_loop(0, NCHUNK, _chunk, 0)
    plsc.subcore_barrier()
    pltpu.sync_copy(osh.at[pl.ds(sid * ZW, ZW)],
                    o2_hbm.at[cid, pl.ds(sid * ZW, ZW)])


def kernel(h, x_s, edge_index, edge_features, W1, b1, W2, b2):
    src = edge_index[0]
    dst = edge_index[1]

    w1a = W1[:, 0:H].T
    w1b = W1[:, H:2 * H].T
    w1c = W1[:, 2 * H:2 * H + S].T
    w1d = W1[:, 2 * H + S:2 * H + 2 * S].T
    w1e = W1[:, 2 * H + 2 * S:].T
    w2bf = W2.reshape(-1).astype(jnp.bfloat16)
    b2pad = jnp.broadcast_to(b2, (16,)).astype(jnp.float32)

    p, q = pl.pallas_call(
        _pq_body,
        out_shape=(
            jax.ShapeDtypeStruct((N, WID), jnp.bfloat16),
            jax.ShapeDtypeStruct((N, WID), jnp.bfloat16),
        ),
    )(h, x_s, w1a, w1b, w1c, w1d, b1)

    r = pl.pallas_call(
        _r_body,
        grid=(E // RBLK,),
        in_specs=[
            pl.BlockSpec((RBLK, EF), lambda i: (i, 0)),
            pl.BlockSpec((EF, WID), lambda i: (0, 0)),
        ],
        out_specs=pl.BlockSpec((RBLK, WID), lambda i: (i, 0)),
        out_shape=jax.ShapeDtypeStruct((E, WID), jnp.bfloat16),
    )(edge_features, w1e)

    exp_s, z2 = _score_kernel(src, dst, p, q, r, w2bf, b2pad)
    weights, o2 = _agg_kernel(src, dst, exp_s, z2, h)

    agg = pl.pallas_call(
        _final_body,
        out_shape=jax.ShapeDtypeStruct((N, H), jnp.float32),
    )(o2)
    return (agg, weights)


# speedup vs baseline: 6.8463x; 1.2043x over previous
import functools
import math

import jax
import jax.numpy as jnp
from jax import lax
from jax.experimental import pallas as pl
from jax.experimental.pallas import tpu as pltpu
from jax.experimental.pallas import tpu_sc as plsc

N = 10000
E = 320000
H = 128
S = 128
EF = 16
WID = 2 * H
IN_SIZE = 2 * H + 2 * S + EF

NC = 2
NS = 16
NW = NC * NS
EW = E // NW
CA = 80
NCHUNK = EW // CA
NGRP = CA // 16
NZ = 10240
ZW = NZ // NS
RBLK = 8000
INV_TEMP = 1.0 / math.sqrt(float(H))

_mesh = plsc.VectorSubcoreMesh(core_axis_name="c", subcore_axis_name="s")
_sc_params = pltpu.CompilerParams(
    use_tc_tiling_on_sc=False, needs_layout_passes=False)


def _prep_body(ef_ref, w1e_ref, h_ref, xs_ref, w1a_ref, w1b_ref, w1c_ref,
               w1d_ref, b1_ref, r_ref, p_ref, q_ref):
    @pl.when(pl.program_id(0) == 0)
    def _():
        h = h_ref[...]
        xs = xs_ref[...]
        p_ref[...] = (
            jnp.dot(h, w1a_ref[...], preferred_element_type=jnp.float32)
            + jnp.dot(xs, w1c_ref[...], preferred_element_type=jnp.float32)
        ).astype(jnp.bfloat16)
        q_ref[...] = (
            jnp.dot(h, w1b_ref[...], preferred_element_type=jnp.float32)
            + jnp.dot(xs, w1d_ref[...], preferred_element_type=jnp.float32)
            + b1_ref[...]
        ).astype(jnp.bfloat16)

    r_ref[...] = jnp.dot(
        ef_ref[...], w1e_ref[...],
        preferred_element_type=jnp.float32).astype(jnp.bfloat16)


def _final_body(o2_ref, out_ref):
    out_ref[...] = o2_ref[0, :N, :] + o2_ref[1, :N, :]


@functools.partial(
    pl.kernel,
    out_type=(
        jax.ShapeDtypeStruct((E,), jnp.float32),
        jax.ShapeDtypeStruct((NC, NZ), jnp.float32),
    ),
    mesh=_mesh,
    compiler_params=_sc_params,
    scratch_types=[
        pltpu.VMEM((2, CA), jnp.int32),
        pltpu.VMEM((2, CA), jnp.int32),
        pltpu.VMEM((CA, WID), jnp.bfloat16),
        pltpu.VMEM((CA, WID), jnp.bfloat16),
        pltpu.VMEM((CA, WID), jnp.bfloat16),
        pltpu.VMEM((CA, WID), jnp.bfloat16),
        pltpu.VMEM((CA, WID), jnp.bfloat16),
        pltpu.VMEM((CA, WID), jnp.bfloat16),
        pltpu.VMEM((WID,), jnp.bfloat16),
        pltpu.VMEM((16,), jnp.float32),
        pltpu.VMEM((CA,), jnp.float32),
        pltpu.VMEM((NZ,), jnp.float32),
        pltpu.VMEM((ZW,), jnp.float32),
        pltpu.VMEM((ZW,), jnp.float32),
        pltpu.VMEM_SHARED((NS, NZ), jnp.float32),
        pltpu.SemaphoreType.DMA,
        pltpu.SemaphoreType.DMA,
        pltpu.SemaphoreType.DMA,
        pltpu.SemaphoreType.DMA,
        pltpu.SemaphoreType.DMA,
        pltpu.SemaphoreType.DMA,
    ],
)
def _score_kernel(src_hbm, dst_hbm, p_hbm, q_hbm, r_hbm, w2_hbm, b2_hbm,
                  exp_hbm, z2_hbm,
                  sidx, didx, pb0, pb1, qb0, qb1, rb0, rb1, w2v, b2v,
                  expb, zloc, zacc, ztmp, zsh,
                  sp0, sq0, sr0, sp1, sq1, sr1):
    cid = lax.axis_index("c")
    sid = lax.axis_index("s")
    wid = sid * NC + cid
    base0 = wid * EW

    bufs = ((pb0, qb0, rb0, sp0, sq0, sr0),
            (pb1, qb1, rb1, sp1, sq1, sr1))

    pltpu.sync_copy(w2_hbm, w2v)
    pltpu.sync_copy(b2_hbm, b2v)
    b2s = b2v[...]

    def _zero_zloc(i, carry):
        zloc[pl.ds(i * 16, 16)] = jnp.zeros((16,), jnp.float32)
        return carry
    lax.fori_loop(0, NZ // 16, _zero_zloc, 0)

    def _issue(c, b):
        pbb, qbb, rbb, sp, sq, sr = bufs[b]
        base = base0 + c * CA
        pltpu.sync_copy(src_hbm.at[pl.ds(base, CA)], sidx.at[b])
        pltpu.sync_copy(dst_hbm.at[pl.ds(base, CA)], didx.at[b])
        pltpu.async_copy(p_hbm.at[sidx.at[b]], pbb, sp)
        pltpu.async_copy(q_hbm.at[didx.at[b]], qbb, sq)
        pltpu.async_copy(r_hbm.at[pl.ds(base, CA)], rbb, sr)

    def _compute(c, b):
        pbb, qbb, rbb, sp, sq, sr = bufs[b]
        base = base0 + c * CA
        pltpu.make_async_copy(p_hbm.at[sidx.at[b]], pbb, sp).wait()
        pltpu.make_async_copy(q_hbm.at[didx.at[b]], qbb, sq).wait()
        pltpu.make_async_copy(r_hbm.at[pl.ds(base, CA)], rbb, sr).wait()

        lane15 = lax.iota(jnp.int32, 16) == 15

        def _edge(e, ecarry):
            acc0 = jnp.zeros((16,), jnp.float32)
            acc1 = jnp.zeros((16,), jnp.float32)
            bzero = jnp.zeros((32,), jnp.bfloat16)
            for j in range(WID // 32):
                sl = pl.ds(j * 32, 32)
                u = pbb[e, sl] + qbb[e, sl] + rbb[e, sl]
                hv = jnp.maximum(u, bzero) * w2v[sl]
                t0, t1 = plsc.unpack(hv, format=plsc.PackFormat.INTERLEAVED)
                acc0 = acc0 + t0
                acc1 = acc1 + t1
            sv = plsc.cumsum(acc0 + acc1)
            plsc.store_scatter(expb, [jnp.full((16,), e, jnp.int32)], sv,
                               mask=lane15)
            return ecarry

        lax.fori_loop(0, CA, _edge, 0)

        def _group(g, gcarry):
            sl = pl.ds(g * 16, 16)
            raw = expb[sl] + b2s
            raw = jnp.where(raw >= 0.0, raw, 0.01 * raw)
            es = jnp.exp(raw * INV_TEMP)
            expb[sl] = es
            didx_g = didx[b, sl]
            plsc.addupdate_scatter(zloc, [didx_g], es)
            return gcarry

        lax.fori_loop(0, NGRP, _group, 0)
        pltpu.sync_copy(expb, exp_hbm.at[pl.ds(base, CA)])

    _issue(0, 0)

    def _pair(pi, carry):
        c0 = pi * 2

        @pl.when(c0 + 1 < NCHUNK)
        def _():
            _issue(c0 + 1, 1)
        _compute(c0, 0)

        @pl.when(c0 + 2 < NCHUNK)
        def _():
            _issue(c0 + 2, 0)

        @pl.when(c0 + 1 < NCHUNK)
        def _():
            _compute(c0 + 1, 1)
        return carry

    lax.fori_loop(0, (NCHUNK + 1) // 2, _pair, 0)

    pltpu.sync_copy(zloc, zsh.at[sid])
    plsc.subcore_barrier()
    off = sid * ZW

    def _zero_zacc(i, carry):
        zacc[pl.ds(i * 16, 16)] = jnp.zeros((16,), jnp.float32)
        return carry
    lax.fori_loop(0, ZW // 16, _zero_zacc, 0)

    def _reduce(j, carry):
        pltpu.sync_copy(zsh.at[j, pl.ds(off, ZW)], ztmp)

        def _acc(i, c2):
            sl = pl.ds(i * 16, 16)
            zacc[sl] = zacc[sl] + ztmp[sl]
            return c2
        lax.fori_loop(0, ZW // 16, _acc, 0)
        return carry
    lax.fori_loop(0, NS, _reduce, 0)
    pltpu.sync_copy(zacc, z2_hbm.at[cid, pl.ds(off, ZW)])


@functools.partial(
    pl.kernel,
    out_type=(
        jax.ShapeDtypeStruct((E,), jnp.float32),
        jax.ShapeDtypeStruct((NC, NZ, H), jnp.float32),
    ),
    mesh=_mesh,
    compiler_params=_sc_params,
    scratch_types=[
        pltpu.VMEM((2, CA), jnp.int32),
        pltpu.VMEM((2, CA), jnp.int32),
        pltpu.VMEM((CA, H), jnp.float32),
        pltpu.VMEM((CA, H), jnp.float32),
        pltpu.VMEM((2, CA), jnp.float32),
        pltpu.VMEM((CA,), jnp.float32),
        pltpu.VMEM((NZ,), jnp.float32),
        pltpu.VMEM((NZ,), jnp.float32),
        pltpu.VMEM_SHARED((NZ, H), jnp.float32),
        pltpu.SemaphoreType.DMA,
        pltpu.SemaphoreType.DMA,
    ],
)
def _agg_kernel(src_hbm, dst_hbm, exp_hbm, z2_hbm, h_hbm,
                w_hbm, o2_hbm,
                sidx, didx, hb0, hb1, eb, wb, za, zb, osh, sh0, sh1):
    cid = lax.axis_index("c")
    sid = lax.axis_index("s")
    wid = sid * NC + cid
    base0 = wid * EW

    hbufs = ((hb0, sh0), (hb1, sh1))

    pltpu.sync_copy(z2_hbm.at[0], za)
    pltpu.sync_copy(z2_hbm.at[1], zb)

    def _zsum(i, carry):
        sl = pl.ds(i * 16, 16)
        za[sl] = za[sl] + zb[sl]
        return carry
    lax.fori_loop(0, NZ // 16, _zsum, 0)

    def _zero_hb(e, carry):
        for j in range(H // 16):
            hb0[e, pl.ds(j * 16, 16)] = jnp.zeros((16,), jnp.float32)
        return carry
    lax.fori_loop(0, CA, _zero_hb, 0)

    def _zero_osh(j, carry):
        pltpu.sync_copy(hb0, osh.at[pl.ds(sid * ZW + j * CA, CA)])
        return carry
    lax.fori_loop(0, ZW // CA, _zero_osh, 0)
    plsc.subcore_barrier()

    def _issue(c, b):
        hbb, sh = hbufs[b]
        base = base0 + c * CA
        pltpu.sync_copy(src_hbm.at[pl.ds(base, CA)], sidx.at[b])
        pltpu.sync_copy(dst_hbm.at[pl.ds(base, CA)], didx.at[b])
        pltpu.sync_copy(exp_hbm.at[pl.ds(base, CA)], eb.at[b])
        pltpu.async_copy(h_hbm.at[sidx.at[b]], hbb, sh)

    def _compute(c, b):
        hbb, sh = hbufs[b]
        base = base0 + c * CA

        def _wgrp(g, gcarry):
            sl = pl.ds(g * 16, 16)
            didx_g = didx[b, sl]
            zv = plsc.load_gather(za, [didx_g])
            wb[sl] = eb[b, sl] / (zv + 1e-9)
            return gcarry
        lax.fori_loop(0, NGRP, _wgrp, 0)
        pltpu.make_async_copy(h_hbm.at[sidx.at[b]], hbb, sh).wait()

        def _scale(e, scarry):
            we = plsc.load_gather(wb, [jnp.full((16,), e, jnp.int32)])
            for j in range(H // 16):
                sl = pl.ds(j * 16, 16)
                hbb[e, sl] = hbb[e, sl] * we
            return scarry
        lax.fori_loop(0, CA, _scale, 0)

        pltpu.sync_copy(wb, w_hbm.at[pl.ds(base, CA)])
        pltpu.sync_copy(hbb, osh.at[didx.at[b]], add=True)

    _issue(0, 0)

    def _pair(pi, carry):
        c0 = pi * 2

        @pl.when(c0 + 1 < NCHUNK)
        def _():
            _issue(c0 + 1, 1)
        _compute(c0, 0)

        @pl.when(c0 + 2 < NCHUNK)
        def _():
            _issue(c0 + 2, 0)

        @pl.when(c0 + 1 < NCHUNK)
        def _():
            _compute(c0 + 1, 1)
        return carry

    lax.fori_loop(0, (NCHUNK + 1) // 2, _pair, 0)
    plsc.subcore_barrier()
    pltpu.sync_copy(osh.at[pl.ds(sid * ZW, ZW)],
                    o2_hbm.at[cid, pl.ds(sid * ZW, ZW)])


def kernel(h, x_s, edge_index, edge_features, W1, b1, W2, b2):
    src = edge_index[0]
    dst = edge_index[1]

    w1a = W1[:, 0:H].T
    w1b = W1[:, H:2 * H].T
    w1c = W1[:, 2 * H:2 * H + S].T
    w1d = W1[:, 2 * H + S:2 * H + 2 * S].T
    w1e = W1[:, 2 * H + 2 * S:].T
    w2bf = W2.reshape(-1).astype(jnp.bfloat16)
    b2pad = jnp.broadcast_to(b2, (16,)).astype(jnp.float32)

    full = lambda i: (0, 0)
    r, p, q = pl.pallas_call(
        _prep_body,
        grid=(E // RBLK,),
        in_specs=[
            pl.BlockSpec((RBLK, EF), lambda i: (i, 0)),
            pl.BlockSpec((EF, WID), full),
            pl.BlockSpec((N, H), full),
            pl.BlockSpec((N, S), full),
            pl.BlockSpec((H, WID), full),
            pl.BlockSpec((H, WID), full),
            pl.BlockSpec((S, WID), full),
            pl.BlockSpec((S, WID), full),
            pl.BlockSpec((WID,), lambda i: (0,)),
        ],
        out_specs=(
            pl.BlockSpec((RBLK, WID), lambda i: (i, 0)),
            pl.BlockSpec((N, WID), full),
            pl.BlockSpec((N, WID), full),
        ),
        out_shape=(
            jax.ShapeDtypeStruct((E, WID), jnp.bfloat16),
            jax.ShapeDtypeStruct((N, WID), jnp.bfloat16),
            jax.ShapeDtypeStruct((N, WID), jnp.bfloat16),
        ),
    )(edge_features, w1e, h, x_s, w1a, w1b, w1c, w1d, b1)

    exp_s, z2 = _score_kernel(src, dst, p, q, r, w2bf, b2pad)
    weights, o2 = _agg_kernel(src, dst, exp_s, z2, h)

    agg = pl.pallas_call(
        _final_body,
        out_shape=jax.ShapeDtypeStruct((N, H), jnp.float32),
    )(o2)
    return (agg, weights)
